# baseline (speedup 1.0000x reference)
import functools

import jax
import jax.numpy as jnp
from jax import lax
from jax.experimental import pallas as pl
from jax.experimental.pallas import tpu as pltpu
from jax.experimental.pallas import tpu_sc as plsc

STATE_DIM = 256
HID = 256
BATCH = 64
BAG = 2048
N_SELECT = 32
ROWS = BATCH * BAG
MLP_BLK = 2048
W2PAD = 128


def _mlp_body(x_ref, w1_ref, b1_ref, w2_ref, out_ref):
    xb = x_ref[...].astype(jnp.bfloat16)
    h = lax.dot_general(xb, w1_ref[...], (((1,), (0,)), ((), ())),
                        preferred_element_type=jnp.float32)
    h = jnp.maximum(h + b1_ref[...], 0.0).astype(jnp.bfloat16)
    s = lax.dot_general(h, w2_ref[...], (((1,), (0,)), ((), ())),
                        preferred_element_type=jnp.float32)
    out_ref[...] = s[:, 0:1]


def _topk_body(p_ref, act_ref, flat_ref, lp_ref, cl_ref):
    cl_ref[...] = p_ref[...]
    col = lax.broadcasted_iota(jnp.int32, (BATCH, BAG), 1)
    kcol = lax.broadcasted_iota(jnp.int32, (BATCH, N_SELECT), 1)

    def body(k, carry):
        acts, logsum = carry
        c = cl_ref[...]
        m = jnp.max(c, axis=1, keepdims=True)
        isel = jnp.min(jnp.where(c == m, col, BAG), axis=1, keepdims=True)
        logsum = logsum + jnp.log(m)
        cl_ref[...] = jnp.where(col == isel, -1.0, c)
        acts = jnp.where(kcol == k, isel, acts)
        return acts, logsum

    acts, logsum = lax.fori_loop(
        0, N_SELECT, body,
        (jnp.zeros((BATCH, N_SELECT), jnp.int32),
         jnp.zeros((BATCH, 1), jnp.float32)))
    act_ref[...] = acts
    row = lax.broadcasted_iota(jnp.int32, (BATCH, N_SELECT), 0)
    flat_ref[...] = acts + row * BAG
    lp_ref[...] = logsum


def _mlp_scores(xf, W1T16, b1, W2P16):
    grid = ROWS // MLP_BLK
    return pl.pallas_call(
        _mlp_body,
        grid=(grid,),
        in_specs=[
            pl.BlockSpec((MLP_BLK, STATE_DIM), lambda i: (i, 0)),
            pl.BlockSpec((STATE_DIM, HID), lambda i: (0, 0)),
            pl.BlockSpec((1, HID), lambda i: (0, 0)),
            pl.BlockSpec((HID, W2PAD), lambda i: (0, 0)),
        ],
        out_specs=pl.BlockSpec((MLP_BLK, 1), lambda i: (i, 0)),
        out_shape=jax.ShapeDtypeStruct((ROWS, 1), jnp.float32),
    )(xf, W1T16, b1.reshape(1, HID), W2P16)


def _topk(clamped):
    return pl.pallas_call(
        _topk_body,
        out_shape=[
            jax.ShapeDtypeStruct((BATCH, N_SELECT), jnp.int32),
            jax.ShapeDtypeStruct((BATCH, N_SELECT), jnp.int32),
            jax.ShapeDtypeStruct((BATCH, 1), jnp.float32),
        ],
        scratch_shapes=[pltpu.VMEM((BATCH, BAG), jnp.float32)],
    )(clamped)


def _sc_gather(xf, flat_idx):
    info = plsc.get_sparse_core_info()
    nc, ns = info.num_cores, info.num_subcores
    nw = nc * ns
    b_per_w = (BATCH * N_SELECT) // nw

    @functools.partial(
        pl.kernel,
        mesh=plsc.VectorSubcoreMesh(core_axis_name="c", subcore_axis_name="s"),
        out_type=jax.ShapeDtypeStruct((BATCH * N_SELECT, STATE_DIM),
                                      jnp.float32),
        scratch_types=[
            pltpu.VMEM((b_per_w,), jnp.int32),
            pltpu.VMEM((b_per_w, STATE_DIM), jnp.float32),
            pltpu.SemaphoreType.DMA,
        ],
    )
    def gather(table_hbm, idx_hbm, out_hbm, idx_v, rows_v, sem):
        wid = lax.axis_index("s") * nc + lax.axis_index("c")
        base = wid * b_per_w
        pltpu.sync_copy(idx_hbm.at[pl.ds(base, b_per_w)], idx_v)
        pltpu.async_copy(table_hbm.at[idx_v], rows_v, sem).wait()
        pltpu.sync_copy(rows_v, out_hbm.at[pl.ds(base, b_per_w)])

    return gather(xf, flat_idx)


def kernel(x, W1, b1, W2, b2):
    B, N, D = x.shape
    xf = x.reshape(B * N, D)
    W1T16 = W1.T.astype(jnp.bfloat16)
    W2P16 = jnp.zeros((HID, W2PAD), jnp.bfloat16).at[:, 0].set(
        W2[0].astype(jnp.bfloat16))
    s_raw = _mlp_scores(xf, W1T16, b1, W2P16)
    probs = jax.nn.sigmoid(s_raw.reshape(B, N) + b2[0])
    clamped = jnp.clip(probs, 1e-8, 1.0)
    action, flat, lp = _topk(clamped)
    sel = _sc_gather(xf, flat.reshape(B * N_SELECT))
    return (probs, action, lp.reshape(B), sel.reshape(B, N_SELECT, D))


# speedup vs baseline: 1.1679x; 1.1679x over previous
import functools

import jax
import jax.numpy as jnp
from jax import lax
from jax.experimental import pallas as pl
from jax.experimental.pallas import tpu as pltpu
from jax.experimental.pallas import tpu_sc as plsc

STATE_DIM = 256
HID = 256
BATCH = 64
BAG = 2048
N_SELECT = 32
ROWS = BATCH * BAG
MLP_BLK = 4096
W2PAD = 128


def _mlp_body(x_ref, w1_ref, b1_ref, w2_ref, out_ref):
    h = lax.dot_general(x_ref[...], w1_ref[...], (((1,), (0,)), ((), ())),
                        preferred_element_type=jnp.float32)
    h = jnp.maximum(h + b1_ref[...], 0.0)
    s = lax.dot_general(h, w2_ref[...], (((1,), (0,)), ((), ())),
                        preferred_element_type=jnp.float32)
    out_ref[...] = s[:, 0:1]


def _topk_body(p_ref, act_ref, flat_ref, lp_ref, cl_ref):
    cl_ref[...] = p_ref[...]
    col = lax.broadcasted_iota(jnp.int32, (BATCH, BAG), 1)
    kcol = lax.broadcasted_iota(jnp.int32, (BATCH, N_SELECT), 1)

    def body(k, carry):
        acts, logsum = carry
        c = cl_ref[...]
        m = jnp.max(c, axis=1, keepdims=True)
        isel = jnp.min(jnp.where(c == m, col, BAG), axis=1, keepdims=True)
        logsum = logsum + jnp.log(m)
        cl_ref[...] = jnp.where(col == isel, -1.0, c)
        acts = jnp.where(kcol == k, isel, acts)
        return acts, logsum

    acts, logsum = lax.fori_loop(
        0, N_SELECT, body,
        (jnp.zeros((BATCH, N_SELECT), jnp.int32),
         jnp.zeros((BATCH, 1), jnp.float32)))
    act_ref[...] = acts
    row = lax.broadcasted_iota(jnp.int32, (BATCH, N_SELECT), 0)
    flat_ref[...] = acts + row * BAG
    lp_ref[...] = logsum


def _mlp_scores(xf, W1T16, b1, W2P16):
    grid = ROWS // MLP_BLK
    return pl.pallas_call(
        _mlp_body,
        grid=(grid,),
        in_specs=[
            pl.BlockSpec((MLP_BLK, STATE_DIM), lambda i: (i, 0)),
            pl.BlockSpec((STATE_DIM, HID), lambda i: (0, 0)),
            pl.BlockSpec((1, HID), lambda i: (0, 0)),
            pl.BlockSpec((HID, W2PAD), lambda i: (0, 0)),
        ],
        out_specs=pl.BlockSpec((MLP_BLK, 1), lambda i: (i, 0)),
        out_shape=jax.ShapeDtypeStruct((ROWS, 1), jnp.float32),
        compiler_params=pltpu.CompilerParams(
            dimension_semantics=("parallel",)),
    )(xf, W1T16, b1.reshape(1, HID), W2P16)


def _topk(clamped):
    return pl.pallas_call(
        _topk_body,
        out_shape=[
            jax.ShapeDtypeStruct((BATCH, N_SELECT), jnp.int32),
            jax.ShapeDtypeStruct((BATCH, N_SELECT), jnp.int32),
            jax.ShapeDtypeStruct((BATCH, 1), jnp.float32),
        ],
        scratch_shapes=[pltpu.VMEM((BATCH, BAG), jnp.float32)],
    )(clamped)


def _sc_gather(xf, flat_idx):
    info = plsc.get_sparse_core_info()
    nc, ns = info.num_cores, info.num_subcores
    nw = nc * ns
    b_per_w = (BATCH * N_SELECT) // nw

    @functools.partial(
        pl.kernel,
        mesh=plsc.VectorSubcoreMesh(core_axis_name="c", subcore_axis_name="s"),
        out_type=jax.ShapeDtypeStruct((BATCH * N_SELECT, STATE_DIM),
                                      jnp.float32),
        scratch_types=[
            pltpu.VMEM((b_per_w,), jnp.int32),
            pltpu.VMEM((b_per_w, STATE_DIM), jnp.float32),
            pltpu.SemaphoreType.DMA,
        ],
    )
    def gather(table_hbm, idx_hbm, out_hbm, idx_v, rows_v, sem):
        wid = lax.axis_index("s") * nc + lax.axis_index("c")
        base = wid * b_per_w
        pltpu.sync_copy(idx_hbm.at[pl.ds(base, b_per_w)], idx_v)
        pltpu.async_copy(table_hbm.at[idx_v], rows_v, sem).wait()
        pltpu.sync_copy(rows_v, out_hbm.at[pl.ds(base, b_per_w)])

    return gather(xf, flat_idx)


def kernel(x, W1, b1, W2, b2):
    B, N, D = x.shape
    xf = x.reshape(B * N, D)
    W1T16 = W1.T.astype(jnp.bfloat16)
    W2P16 = jnp.zeros((HID, W2PAD), jnp.bfloat16).at[:, 0].set(
        W2[0].astype(jnp.bfloat16))
    s_raw = _mlp_scores(xf, W1T16, b1, W2P16)
    probs = jax.nn.sigmoid(s_raw.reshape(B, N) + b2[0])
    clamped = jnp.clip(probs, 1e-8, 1.0)
    action, flat, lp = _topk(clamped)
    sel = _sc_gather(xf, flat.reshape(B * N_SELECT))
    return (probs, action, lp.reshape(B), sel.reshape(B, N_SELECT, D))


# speedup vs baseline: 1.5639x; 1.3391x over previous
import functools

import jax
import jax.numpy as jnp
from jax import lax
from jax.experimental import pallas as pl
from jax.experimental.pallas import tpu as pltpu
from jax.experimental.pallas import tpu_sc as plsc

STATE_DIM = 256
HID = 256
BATCH = 64
BAG = 2048
N_SELECT = 32
ROWS = BATCH * BAG
MLP_BLK = 4096
W2PAD = 128


def _mlp_body(x_hbm, w1_ref, b1_ref, w2_ref, out_ref, xbuf, sems):
    nb = ROWS // MLP_BLK

    def copy_in(blk, slot):
        return pltpu.make_async_copy(
            x_hbm.at[pl.ds(blk * MLP_BLK, MLP_BLK), :],
            xbuf.at[slot], sems.at[slot])

    copy_in(0, 0).start()

    def step(i, carry):
        slot = lax.rem(i, 2)
        nslot = lax.rem(i + 1, 2)

        @pl.when(i + 1 < nb)
        def _():
            copy_in(i + 1, nslot).start()

        copy_in(i, slot).wait()
        xq = xbuf[slot]
        h = lax.dot_general(xq, w1_ref[...], (((1,), (0,)), ((), ())),
                            preferred_element_type=jnp.float32)
        h = jnp.maximum(h + b1_ref[...], 0.0)
        s = lax.dot_general(w2_ref[...], h, (((1,), (1,)), ((), ())),
                            preferred_element_type=jnp.float32)
        out_ref[pl.ds(i, 1), :] = s[0:1, :]
        return carry

    lax.fori_loop(0, nb, step, 0)


def _topk_body(p_ref, act_ref, flat_ref, lp_ref, cl_ref):
    cl_ref[...] = p_ref[...]
    col = lax.broadcasted_iota(jnp.int32, (BATCH, BAG), 1)
    kcol = lax.broadcasted_iota(jnp.int32, (BATCH, N_SELECT), 1)

    def body(k, carry):
        acts, logsum = carry
        c = cl_ref[...]
        m = jnp.max(c, axis=1, keepdims=True)
        isel = jnp.min(jnp.where(c == m, col, BAG), axis=1, keepdims=True)
        logsum = logsum + jnp.log(m)
        cl_ref[...] = jnp.where(col == isel, -1.0, c)
        acts = jnp.where(kcol == k, isel, acts)
        return acts, logsum

    acts, logsum = lax.fori_loop(
        0, N_SELECT, body,
        (jnp.zeros((BATCH, N_SELECT), jnp.int32),
         jnp.zeros((BATCH, 1), jnp.float32)))
    act_ref[...] = acts
    row = lax.broadcasted_iota(jnp.int32, (BATCH, N_SELECT), 0)
    flat_ref[...] = acts + row * BAG
    lp_ref[...] = logsum


def _mlp_scores(xf, W1T16, b1, W2P16):
    return pl.pallas_call(
        _mlp_body,
        in_specs=[
            pl.BlockSpec(memory_space=pl.ANY),
            pl.BlockSpec((STATE_DIM, HID), lambda: (0, 0)),
            pl.BlockSpec((1, HID), lambda: (0, 0)),
            pl.BlockSpec((W2PAD, HID), lambda: (0, 0)),
        ],
        out_specs=pl.BlockSpec((ROWS // MLP_BLK, MLP_BLK), lambda: (0, 0)),
        out_shape=jax.ShapeDtypeStruct((ROWS // MLP_BLK, MLP_BLK),
                                       jnp.float32),
        scratch_shapes=[
            pltpu.VMEM((2, MLP_BLK, STATE_DIM), jnp.float32),
            pltpu.SemaphoreType.DMA((2,)),
        ],
    )(xf, W1T16, b1.reshape(1, HID), W2P16)


def _topk(clamped):
    return pl.pallas_call(
        _topk_body,
        out_shape=[
            jax.ShapeDtypeStruct((BATCH, N_SELECT), jnp.int32),
            jax.ShapeDtypeStruct((BATCH, N_SELECT), jnp.int32),
            jax.ShapeDtypeStruct((BATCH, 1), jnp.float32),
        ],
        scratch_shapes=[pltpu.VMEM((BATCH, BAG), jnp.float32)],
    )(clamped)


def _sc_gather(xf, flat_idx):
    info = plsc.get_sparse_core_info()
    nc, ns = info.num_cores, info.num_subcores
    nw = nc * ns
    b_per_w = (BATCH * N_SELECT) // nw

    @functools.partial(
        pl.kernel,
        mesh=plsc.VectorSubcoreMesh(core_axis_name="c", subcore_axis_name="s"),
        out_type=jax.ShapeDtypeStruct((BATCH * N_SELECT, STATE_DIM),
                                      jnp.float32),
        scratch_types=[
            pltpu.VMEM((b_per_w,), jnp.int32),
            pltpu.VMEM((b_per_w, STATE_DIM), jnp.float32),
            pltpu.SemaphoreType.DMA,
        ],
    )
    def gather(table_hbm, idx_hbm, out_hbm, idx_v, rows_v, sem):
        wid = lax.axis_index("s") * nc + lax.axis_index("c")
        base = wid * b_per_w
        pltpu.sync_copy(idx_hbm.at[pl.ds(base, b_per_w)], idx_v)
        pltpu.async_copy(table_hbm.at[idx_v], rows_v, sem).wait()
        pltpu.sync_copy(rows_v, out_hbm.at[pl.ds(base, b_per_w)])

    return gather(xf, flat_idx)


def kernel(x, W1, b1, W2, b2):
    B, N, D = x.shape
    xf = x.reshape(B * N, D)
    W1T16 = W1.T.astype(jnp.bfloat16)
    W2P16 = jnp.zeros((W2PAD, HID), jnp.bfloat16).at[0].set(
        W2[0].astype(jnp.bfloat16))
    s_raw = _mlp_scores(xf, W1T16, b1, W2P16)
    probs = jax.nn.sigmoid(s_raw.reshape(B, N) + b2[0])
    clamped = jnp.clip(probs, 1e-8, 1.0)
    action, flat, lp = _topk(clamped)
    sel = _sc_gather(xf, flat.reshape(B * N_SELECT))
    return (probs, action, lp.reshape(B), sel.reshape(B, N_SELECT, D))


# speedup vs baseline: 1.7611x; 1.1261x over previous
import functools

import jax
import jax.numpy as jnp
from jax import lax
from jax.experimental import pallas as pl
from jax.experimental.pallas import tpu as pltpu
from jax.experimental.pallas import tpu_sc as plsc

STATE_DIM = 256
HID = 256
BATCH = 64
BAG = 2048
N_SELECT = 32
ROWS = BATCH * BAG
MLP_BLK = 4096
NBUF = 4
W2PAD = 128


def _mlp_body(x_hbm, w1_ref, b1_ref, w2_ref, out_ref, xbuf, sems):
    nb = ROWS // MLP_BLK

    def copy_in(blk, slot):
        return pltpu.make_async_copy(
            x_hbm.at[pl.ds(blk * MLP_BLK, MLP_BLK), :],
            xbuf.at[slot], sems.at[slot])

    for b in range(NBUF - 1):
        copy_in(b, b).start()

    def step(i, carry):
        slot = lax.rem(i, NBUF)
        nslot = lax.rem(i + NBUF - 1, NBUF)

        @pl.when(i + NBUF - 1 < nb)
        def _():
            copy_in(i + NBUF - 1, nslot).start()

        copy_in(i, slot).wait()
        xq = xbuf[slot]
        h = lax.dot_general(xq, w1_ref[...], (((1,), (0,)), ((), ())),
                            preferred_element_type=jnp.float32)
        h = jnp.maximum(h + b1_ref[...], 0.0)
        s = lax.dot_general(w2_ref[...], h, (((1,), (1,)), ((), ())),
                            preferred_element_type=jnp.float32)
        out_ref[pl.ds(i, 1), :] = s[0:1, :]
        return carry

    lax.fori_loop(0, nb, step, 0)


def _topk_body(p_ref, act_ref, flat_ref, lp_ref, cl_ref):
    cl_ref[...] = p_ref[...]
    col = lax.broadcasted_iota(jnp.int32, (BATCH, BAG), 1)
    kcol = lax.broadcasted_iota(jnp.int32, (BATCH, N_SELECT), 1)

    def body(k, carry):
        acts, logsum = carry
        c = cl_ref[...]
        m = jnp.max(c, axis=1, keepdims=True)
        isel = jnp.min(jnp.where(c == m, col, BAG), axis=1, keepdims=True)
        logsum = logsum + jnp.log(m)
        cl_ref[...] = jnp.where(col == isel, -1.0, c)
        acts = jnp.where(kcol == k, isel, acts)
        return acts, logsum

    acts, logsum = lax.fori_loop(
        0, N_SELECT, body,
        (jnp.zeros((BATCH, N_SELECT), jnp.int32),
         jnp.zeros((BATCH, 1), jnp.float32)))
    act_ref[...] = acts
    row = lax.broadcasted_iota(jnp.int32, (BATCH, N_SELECT), 0)
    flat_ref[...] = acts + row * BAG
    lp_ref[...] = logsum


def _mlp_scores(xf, W1T16, b1, W2P16):
    return pl.pallas_call(
        _mlp_body,
        in_specs=[
            pl.BlockSpec(memory_space=pl.ANY),
            pl.BlockSpec((STATE_DIM, HID), lambda: (0, 0)),
            pl.BlockSpec((1, HID), lambda: (0, 0)),
            pl.BlockSpec((W2PAD, HID), lambda: (0, 0)),
        ],
        out_specs=pl.BlockSpec((ROWS // MLP_BLK, MLP_BLK), lambda: (0, 0)),
        out_shape=jax.ShapeDtypeStruct((ROWS // MLP_BLK, MLP_BLK),
                                       jnp.float32),
        scratch_shapes=[
            pltpu.VMEM((NBUF, MLP_BLK, STATE_DIM), jnp.float32),
            pltpu.SemaphoreType.DMA((NBUF,)),
        ],
    )(xf, W1T16, b1.reshape(1, HID), W2P16)


def _topk(clamped):
    return pl.pallas_call(
        _topk_body,
        out_shape=[
            jax.ShapeDtypeStruct((BATCH, N_SELECT), jnp.int32),
            jax.ShapeDtypeStruct((BATCH, N_SELECT), jnp.int32),
            jax.ShapeDtypeStruct((BATCH, 1), jnp.float32),
        ],
        scratch_shapes=[pltpu.VMEM((BATCH, BAG), jnp.float32)],
    )(clamped)


def _sc_gather(xf, flat_idx):
    info = plsc.get_sparse_core_info()
    nc, ns = info.num_cores, info.num_subcores
    nw = nc * ns
    b_per_w = (BATCH * N_SELECT) // nw

    @functools.partial(
        pl.kernel,
        mesh=plsc.VectorSubcoreMesh(core_axis_name="c", subcore_axis_name="s"),
        out_type=jax.ShapeDtypeStruct((BATCH * N_SELECT, STATE_DIM),
                                      jnp.float32),
        scratch_types=[
            pltpu.VMEM((b_per_w,), jnp.int32),
            pltpu.VMEM((b_per_w, STATE_DIM), jnp.float32),
            pltpu.SemaphoreType.DMA,
        ],
    )
    def gather(table_hbm, idx_hbm, out_hbm, idx_v, rows_v, sem):
        wid = lax.axis_index("s") * nc + lax.axis_index("c")
        base = wid * b_per_w
        pltpu.sync_copy(idx_hbm.at[pl.ds(base, b_per_w)], idx_v)
        pltpu.async_copy(table_hbm.at[idx_v], rows_v, sem).wait()
        pltpu.sync_copy(rows_v, out_hbm.at[pl.ds(base, b_per_w)])

    return gather(xf, flat_idx)


def kernel(x, W1, b1, W2, b2):
    B, N, D = x.shape
    xf = x.reshape(B * N, D)
    W1T16 = W1.T.astype(jnp.bfloat16)
    W2P16 = jnp.zeros((W2PAD, HID), jnp.bfloat16).at[0].set(
        W2[0].astype(jnp.bfloat16))
    s_raw = _mlp_scores(xf, W1T16, b1, W2P16)
    probs = jax.nn.sigmoid(s_raw.reshape(B, N) + b2[0])
    clamped = jnp.clip(probs, 1e-8, 1.0)
    action, flat, lp = _topk(clamped)
    sel = _sc_gather(xf, flat.reshape(B * N_SELECT))
    return (probs, action, lp.reshape(B), sel.reshape(B, N_SELECT, D))
